# baseline (speedup 1.0000x reference)
import functools

import jax
import jax.numpy as jnp
from jax.experimental import pallas as pl
from jax.experimental.pallas import tpu as pltpu
from jax.experimental.pallas import tpu_sc as plsc

CUTOFFS = [2000, 10000, 50000]
SHORTLIST = 2000
NEG_INF = float("-inf")
TB = 256


def _cluster_of(t):
    return ((t >= CUTOFFS[0]).astype(jnp.int32)
            + (t >= CUTOFFS[1]).astype(jnp.int32)
            + (t >= CUTOFFS[2]).astype(jnp.int32))


def _prep1_body(t_ref, pos_ref, cnt_ref, aoff_ref, *, n_tokens):
    tb = pl.program_id(0)
    t_all = t_ref[...]
    cl_all = _cluster_of(t_all)
    cl_row = jnp.transpose(cl_all)
    row0 = tb * TB
    cl_blk = _cluster_of(t_ref[pl.ds(row0, TB), :])
    cols = jax.lax.broadcasted_iota(jnp.int32, (TB, n_tokens), 1)
    rows = row0 + jax.lax.broadcasted_iota(jnp.int32, (TB, n_tokens), 0)
    rank = jnp.sum(jnp.where((cl_row == cl_blk) & (cols < rows), 1, 0),
                   axis=1, keepdims=True)
    cidx = jax.lax.broadcasted_iota(jnp.int32, (8, n_tokens), 0)
    cnt = jnp.sum((cl_row == cidx).astype(jnp.int32), axis=1, keepdims=True)
    rup = cnt
    c_r = jax.lax.broadcasted_iota(jnp.int32, (8, 8), 0)
    c_c = jax.lax.broadcasted_iota(jnp.int32, (8, 8), 1)
    aoff = jnp.sum(jnp.where(c_c < c_r, jnp.transpose(rup), 0), axis=1,
                   keepdims=True)
    cl8 = jax.lax.broadcasted_iota(jnp.int32, (TB, 8), 1)
    my_off = jnp.sum(jnp.where(cl_blk == cl8, jnp.transpose(aoff), 0),
                     axis=1, keepdims=True)
    pos_ref[...] = my_off + rank

    @pl.when(tb == 0)
    def _counts():
        cnt_ref[...] = cnt
        aoff_ref[...] = aoff


def _prep2_body(x_ref, w11_ref, w12_ref, w13_ref, t_ref, pos_ref,
                h1_ref, h2_ref, h3_ref, rel1_ref, rel2_ref, rel3_ref,
                q2_ref, q3_ref, hid1_ref, hid2_ref, hid3_ref, *, n_tokens):
    tb = pl.program_id(0)

    @pl.when(tb == 0)
    def _hidden():
        xb = x_ref[...].astype(jnp.bfloat16)
        for w_ref, hid_ref in ((w11_ref, hid1_ref), (w12_ref, hid2_ref),
                               (w13_ref, hid3_ref)):
            hid_ref[...] = jax.lax.dot_general(
                xb, w_ref[...].astype(jnp.bfloat16), (((1,), (1,)), ((), ())),
                preferred_element_type=jnp.float32).astype(jnp.bfloat16)

    row0 = tb * TB
    p_rows = row0 + jax.lax.broadcasted_iota(jnp.int32, (TB, n_tokens), 0)
    pos_row = jnp.transpose(pos_ref[...])
    s_blk = (pos_row == p_rows)
    s_bf = s_blk.astype(jnp.bfloat16)
    h1_ref[...] = jax.lax.dot_general(
        s_bf, hid1_ref[...], (((1,), (0,)), ((), ())),
        preferred_element_type=jnp.float32).astype(jnp.bfloat16)
    h2_ref[...] = jax.lax.dot_general(
        s_bf, hid2_ref[...], (((1,), (0,)), ((), ())),
        preferred_element_type=jnp.float32).astype(jnp.bfloat16)
    h3_ref[...] = jax.lax.dot_general(
        s_bf, hid3_ref[...], (((1,), (0,)), ((), ())),
        preferred_element_type=jnp.float32).astype(jnp.bfloat16)
    t_row = jnp.transpose(t_ref[...])
    tsv = jnp.sum(jnp.where(s_blk, t_row, 0), axis=1, keepdims=True)
    rel1 = jnp.clip(tsv - CUTOFFS[0], 0, CUTOFFS[1] - CUTOFFS[0] - 1)
    rel2 = jnp.clip(tsv - CUTOFFS[1], 0, CUTOFFS[2] - CUTOFFS[1] - 1)
    rel3 = jnp.clip(tsv - CUTOFFS[2], 0, 100000 - CUTOFFS[2] - 1)
    rel1_ref[...] = rel1
    rel2_ref[...] = rel2 // 2
    rel3_ref[...] = rel3 // 8
    q2_ref[...] = rel2 % 2
    q3_ref[...] = rel3 % 8


def _sc_gather(w21, w22, w23, r1, r2, r3):
    T = r1.shape[0]
    info = plsc.get_sparse_core_info()
    nc = info.num_cores
    nw = nc * info.num_subcores
    bpw = T // nw
    d1, d2, d3 = w21.shape[1], w22.shape[1], w23.shape[1]
    mesh = plsc.VectorSubcoreMesh(core_axis_name="c", subcore_axis_name="s")

    @functools.partial(
        pl.kernel, mesh=mesh,
        out_type=[
            jax.ShapeDtypeStruct((T, d1), jnp.float32),
            jax.ShapeDtypeStruct((T, d2), jnp.float32),
            jax.ShapeDtypeStruct((T, d3), jnp.float32),
        ],
        scratch_types=[
            pltpu.VMEM((bpw,), jnp.int32),
            pltpu.VMEM((bpw,), jnp.int32),
            pltpu.VMEM((bpw,), jnp.int32),
            pltpu.VMEM((bpw, d1), jnp.float32),
            pltpu.VMEM((bpw, d2), jnp.float32),
            pltpu.VMEM((bpw, d3), jnp.float32),
            pltpu.SemaphoreType.DMA,
        ],
    )
    def gather_k(t1, t2, t3, i1, i2, i3, g1, g2, g3,
                 iv1, iv2, iv3, rv1, rv2, rv3, sem):
        wid = jax.lax.axis_index("s") * nc + jax.lax.axis_index("c")
        base = wid * bpw
        for i_hbm, iv, t_hbm, rv, g_hbm in (
                (i1, iv1, t1, rv1, g1), (i2, iv2, t2, rv2, g2),
                (i3, iv3, t3, rv3, g3)):
            pltpu.sync_copy(i_hbm.at[pl.ds(base, bpw)], iv)
            pltpu.async_copy(t_hbm.at[iv], rv, sem).wait()
            pltpu.sync_copy(rv, g_hbm.at[pl.ds(base, bpw)])

    return gather_k(w21, w22, w23, r1, r2, r3)


def _tail_body(oc_ref, h_ref, w2_ref, g_ref, q_ref, out_ref, m_ref, s_ref,
               *, n_cols, blk, n_blk, n_tb, gf):
    cb = pl.program_id(0)
    off = oc_ref[0]
    cnt = oc_ref[1]

    @pl.when(cb == 0)
    def _init():
        m_ref[...] = jnp.full(m_ref.shape, NEG_INF, jnp.float32)
        s_ref[...] = jnp.zeros(s_ref.shape, jnp.float32)

    def _update(masked):
        w2b = w2_ref[...].astype(jnp.bfloat16)
        for j in range(n_tb):
            row0 = j * TB
            active = (row0 < off + cnt) & (row0 + TB > off)

            @pl.when(active)
            def _step(row0=row0):
                hid = h_ref[row0:row0 + TB, :]
                logits = jax.lax.dot_general(
                    hid, w2b, (((1,), (1,)), ((), ())),
                    preferred_element_type=jnp.float32)
                if masked:
                    cols = cb * blk + jax.lax.broadcasted_iota(
                        jnp.int32, logits.shape, 1)
                    lm = jnp.where(cols < n_cols, logits, NEG_INF)
                else:
                    lm = logits
                bm = jnp.max(lm, axis=1, keepdims=True)
                m_old = m_ref[row0:row0 + TB, :]
                m_new = jnp.maximum(m_old, bm)
                s_ref[row0:row0 + TB, :] = (
                    s_ref[row0:row0 + TB, :] * jnp.exp(m_old - m_new)
                    + jnp.sum(jnp.exp(lm - m_new), axis=1, keepdims=True))
                m_ref[row0:row0 + TB, :] = m_new

    @pl.when(cb < n_blk - 1)
    def _full_blocks():
        _update(masked=False)

    @pl.when(cb == n_blk - 1)
    def _last_block():
        _update(masked=True)
        rows = jax.lax.broadcasted_iota(jnp.int32, (m_ref.shape[0], 1), 0)
        valid = (rows >= off) & (rows < off + cnt)
        hf = h_ref[...].astype(jnp.float32)
        if gf == 1:
            tgt = jnp.sum(g_ref[...] * hf, axis=1, keepdims=True)
        else:
            w = hf.shape[1]
            ht = jnp.concatenate([hf] * gf, axis=1)
            seg = jax.lax.broadcasted_iota(jnp.int32, ht.shape, 1) // w
            prod = jnp.where(seg == q_ref[...], g_ref[...] * ht, 0.0)
            tgt = jnp.sum(prod, axis=1, keepdims=True)
        val = tgt - m_ref[...] - jnp.log(s_ref[...])
        out_ref[...] = jnp.where(valid, val, 0.0)


def _tail_logprob(offcnt, h_sorted, w2, g_rows, q, blk, gf):
    T = h_sorted.shape[0]
    n_cols, h = w2.shape
    gw = g_rows.shape[1]
    n_blk = pl.cdiv(n_cols, blk)
    n_tb = T // TB
    body = functools.partial(_tail_body, n_cols=n_cols, blk=blk,
                             n_blk=n_blk, n_tb=n_tb, gf=gf)
    grid_spec = pltpu.PrefetchScalarGridSpec(
        num_scalar_prefetch=1,
        grid=(n_blk,),
        in_specs=[
            pl.BlockSpec((T, h), lambda cb, oc: (0, 0)),
            pl.BlockSpec((blk, h), lambda cb, oc: (cb, 0)),
            pl.BlockSpec((T, gw), lambda cb, oc: (0, 0)),
            pl.BlockSpec((T, 1), lambda cb, oc: (0, 0)),
        ],
        out_specs=pl.BlockSpec((T, 1), lambda cb, oc: (0, 0)),
        scratch_shapes=[
            pltpu.VMEM((T, 1), jnp.float32),
            pltpu.VMEM((T, 1), jnp.float32),
        ],
    )
    return pl.pallas_call(
        body,
        grid_spec=grid_spec,
        out_shape=jax.ShapeDtypeStruct((T, 1), jnp.float32),
    )(offcnt, h_sorted, w2, g_rows, q)


def _head_body(x_ref, hw_ref, t_ref, pos_ref, l1_ref, l2_ref, l3_ref, out_ref,
               loss_ref, acc_ref, *, n_tb, n_tokens):
    tb = pl.program_id(0)
    logits = jax.lax.dot_general(
        x_ref[...].astype(jnp.bfloat16), hw_ref[...].astype(jnp.bfloat16),
        (((1,), (1,)), ((), ())),
        preferred_element_type=jnp.float32)
    cols = jax.lax.broadcasted_iota(jnp.int32, logits.shape, 1)
    m = jnp.max(logits, axis=1, keepdims=True)
    s = jnp.sum(jnp.exp(logits - m), axis=1, keepdims=True)
    t = t_ref[...]
    cl = _cluster_of(t)
    gidx = jnp.where(cl == 0, t, SHORTLIST + cl - 1)
    ht = (jnp.sum(jnp.where(cols == gidx, logits, 0.0), axis=1, keepdims=True)
          - m - jnp.log(s))
    lsum = l1_ref[...] + l2_ref[...] + l3_ref[...]
    p_rows = jax.lax.broadcasted_iota(jnp.int32, (lsum.shape[0], TB), 0)
    s_blk = (jnp.transpose(pos_ref[...]) == p_rows).astype(jnp.float32)
    local = jax.lax.dot_general(
        s_blk, lsum, (((0,), (0,)), ((), ())),
        preferred_element_type=jnp.float32)
    out = local + ht
    out_ref[...] = out

    @pl.when(tb == 0)
    def _z():
        acc_ref[0] = 0.0

    acc_ref[0] += jnp.sum(-out) / n_tokens

    @pl.when(tb == n_tb - 1)
    def _w():
        loss_ref[...] = jnp.full((1, 1), acc_ref[0], jnp.float32)


def kernel(myinput, target, head_W, W1_1, W2_1, W1_2, W2_2, W1_3, W2_3):
    x = myinput
    T, F = x.shape
    t2 = target.astype(jnp.int32).reshape(T, 1)
    n_tb = T // TB

    pos, cnt8, aoff8 = pl.pallas_call(
        functools.partial(_prep1_body, n_tokens=T),
        grid=(n_tb,),
        in_specs=[pl.BlockSpec((T, 1), lambda tb: (0, 0))],
        out_specs=[
            pl.BlockSpec((TB, 1), lambda tb: (tb, 0)),
            pl.BlockSpec((8, 1), lambda tb: (0, 0)),
            pl.BlockSpec((8, 1), lambda tb: (0, 0)),
        ],
        out_shape=[
            jax.ShapeDtypeStruct((T, 1), jnp.int32),
            jax.ShapeDtypeStruct((8, 1), jnp.int32),
            jax.ShapeDtypeStruct((8, 1), jnp.int32),
        ],
    )(t2)

    hs = [W1_1.shape[0], W1_2.shape[0], W1_3.shape[0]]
    h1s, h2s, h3s, rel1, rel2g, rel3g, q2, q3 = pl.pallas_call(
        functools.partial(_prep2_body, n_tokens=T),
        grid=(n_tb,),
        in_specs=[
            pl.BlockSpec((T, F), lambda tb: (0, 0)),
            pl.BlockSpec(W1_1.shape, lambda tb: (0, 0)),
            pl.BlockSpec(W1_2.shape, lambda tb: (0, 0)),
            pl.BlockSpec(W1_3.shape, lambda tb: (0, 0)),
            pl.BlockSpec((T, 1), lambda tb: (0, 0)),
            pl.BlockSpec((T, 1), lambda tb: (0, 0)),
        ],
        out_specs=[
            pl.BlockSpec((TB, hs[0]), lambda tb: (tb, 0)),
            pl.BlockSpec((TB, hs[1]), lambda tb: (tb, 0)),
            pl.BlockSpec((TB, hs[2]), lambda tb: (tb, 0)),
            pl.BlockSpec((TB, 1), lambda tb: (tb, 0)),
            pl.BlockSpec((TB, 1), lambda tb: (tb, 0)),
            pl.BlockSpec((TB, 1), lambda tb: (tb, 0)),
            pl.BlockSpec((TB, 1), lambda tb: (tb, 0)),
            pl.BlockSpec((TB, 1), lambda tb: (tb, 0)),
        ],
        out_shape=[
            jax.ShapeDtypeStruct((T, hs[0]), jnp.bfloat16),
            jax.ShapeDtypeStruct((T, hs[1]), jnp.bfloat16),
            jax.ShapeDtypeStruct((T, hs[2]), jnp.bfloat16),
            jax.ShapeDtypeStruct((T, 1), jnp.int32),
            jax.ShapeDtypeStruct((T, 1), jnp.int32),
            jax.ShapeDtypeStruct((T, 1), jnp.int32),
            jax.ShapeDtypeStruct((T, 1), jnp.int32),
            jax.ShapeDtypeStruct((T, 1), jnp.int32),
        ],
        scratch_shapes=[
            pltpu.VMEM((T, hs[0]), jnp.bfloat16),
            pltpu.VMEM((T, hs[1]), jnp.bfloat16),
            pltpu.VMEM((T, hs[2]), jnp.bfloat16),
        ],
    )(x, W1_1, W1_2, W1_3, t2, pos)

    oc1 = jnp.stack([aoff8[1, 0], cnt8[1, 0]])
    oc2 = jnp.stack([aoff8[2, 0], cnt8[2, 0]])
    oc3 = jnp.stack([aoff8[3, 0], cnt8[3, 0]])

    w22g = W2_2.reshape(-1, 128)
    w23g = W2_3.reshape(-1, 128)
    g1, g2, g3 = _sc_gather(W2_1, w22g, w23g, rel1.reshape(T),
                            rel2g.reshape(T), rel3g.reshape(T))

    l1 = _tail_logprob(oc1, h1s, W2_1, g1, rel1, 2048, 1)
    l2 = _tail_logprob(oc2, h2s, W2_2, g2, q2, 2048, 2)
    l3 = _tail_logprob(oc3, h3s, W2_3, g3, q3, 2048, 8)

    body = functools.partial(_head_body, n_tb=n_tb, n_tokens=T)
    out, loss = pl.pallas_call(
        body,
        grid=(n_tb,),
        in_specs=[
            pl.BlockSpec((TB, F), lambda tb: (tb, 0)),
            pl.BlockSpec(head_W.shape, lambda tb: (0, 0)),
            pl.BlockSpec((TB, 1), lambda tb: (tb, 0)),
            pl.BlockSpec((TB, 1), lambda tb: (tb, 0)),
            pl.BlockSpec((T, 1), lambda tb: (0, 0)),
            pl.BlockSpec((T, 1), lambda tb: (0, 0)),
            pl.BlockSpec((T, 1), lambda tb: (0, 0)),
        ],
        out_specs=[
            pl.BlockSpec((TB, 1), lambda tb: (tb, 0)),
            pl.BlockSpec((1, 1), lambda tb: (0, 0)),
        ],
        out_shape=[
            jax.ShapeDtypeStruct((T, 1), jnp.float32),
            jax.ShapeDtypeStruct((1, 1), jnp.float32),
        ],
        scratch_shapes=[pltpu.SMEM((1,), jnp.float32)],
    )(x, head_W, t2, pos, l1, l2, l3)
    return (out.reshape(T), loss[0, 0])


# speedup vs baseline: 1.2376x; 1.2376x over previous
import functools

import jax
import jax.numpy as jnp
from jax.experimental import pallas as pl
from jax.experimental.pallas import tpu as pltpu
from jax.experimental.pallas import tpu_sc as plsc

CUTOFFS = [2000, 10000, 50000]
SHORTLIST = 2000
NEG_INF = float("-inf")
TB = 256


def _cluster_of(t):
    return ((t >= CUTOFFS[0]).astype(jnp.int32)
            + (t >= CUTOFFS[1]).astype(jnp.int32)
            + (t >= CUTOFFS[2]).astype(jnp.int32))


def _prep1_body(t_ref, pos_ref, cnt_ref, aoff_ref, *, n_tokens):
    tb = pl.program_id(0)
    t_all = t_ref[...]
    cl_all = _cluster_of(t_all)
    cl_row = jnp.transpose(cl_all)
    row0 = tb * TB
    cl_blk = _cluster_of(t_ref[pl.ds(row0, TB), :])
    cols = jax.lax.broadcasted_iota(jnp.int32, (TB, n_tokens), 1)
    rows = row0 + jax.lax.broadcasted_iota(jnp.int32, (TB, n_tokens), 0)
    rank = jnp.sum(jnp.where((cl_row == cl_blk) & (cols < rows), 1, 0),
                   axis=1, keepdims=True)
    cidx = jax.lax.broadcasted_iota(jnp.int32, (8, n_tokens), 0)
    cnt = jnp.sum((cl_row == cidx).astype(jnp.int32), axis=1, keepdims=True)
    rup = cnt
    c_r = jax.lax.broadcasted_iota(jnp.int32, (8, 8), 0)
    c_c = jax.lax.broadcasted_iota(jnp.int32, (8, 8), 1)
    aoff = jnp.sum(jnp.where(c_c < c_r, jnp.transpose(rup), 0), axis=1,
                   keepdims=True)
    cl8 = jax.lax.broadcasted_iota(jnp.int32, (TB, 8), 1)
    my_off = jnp.sum(jnp.where(cl_blk == cl8, jnp.transpose(aoff), 0),
                     axis=1, keepdims=True)
    pos_ref[...] = my_off + rank

    @pl.when(tb == 0)
    def _counts():
        cnt_ref[...] = cnt
        aoff_ref[...] = aoff


def _prep2_body(x_ref, w11_ref, w12_ref, w13_ref, t_ref, pos_ref,
                h1_ref, h2_ref, h3_ref, rel1_ref, rel2_ref, rel3_ref,
                q2_ref, q3_ref, hid1_ref, hid2_ref, hid3_ref, *, n_tokens):
    tb = pl.program_id(0)

    @pl.when(tb == 0)
    def _hidden():
        xb = x_ref[...].astype(jnp.bfloat16)
        for w_ref, hid_ref in ((w11_ref, hid1_ref), (w12_ref, hid2_ref),
                               (w13_ref, hid3_ref)):
            hid_ref[...] = jax.lax.dot_general(
                xb, w_ref[...].astype(jnp.bfloat16), (((1,), (1,)), ((), ())),
                preferred_element_type=jnp.float32).astype(jnp.bfloat16)

    row0 = tb * TB
    p_rows = row0 + jax.lax.broadcasted_iota(jnp.int32, (TB, n_tokens), 0)
    pos_row = jnp.transpose(pos_ref[...])
    s_blk = (pos_row == p_rows)
    s_bf = s_blk.astype(jnp.bfloat16)
    h1_ref[...] = jax.lax.dot_general(
        s_bf, hid1_ref[...], (((1,), (0,)), ((), ())),
        preferred_element_type=jnp.float32).astype(jnp.bfloat16)
    h2_ref[...] = jax.lax.dot_general(
        s_bf, hid2_ref[...], (((1,), (0,)), ((), ())),
        preferred_element_type=jnp.float32).astype(jnp.bfloat16)
    h3_ref[...] = jax.lax.dot_general(
        s_bf, hid3_ref[...], (((1,), (0,)), ((), ())),
        preferred_element_type=jnp.float32).astype(jnp.bfloat16)
    t_row = jnp.transpose(t_ref[...])
    tsv = jnp.sum(jnp.where(s_blk, t_row, 0), axis=1, keepdims=True)
    rel1 = jnp.clip(tsv - CUTOFFS[0], 0, CUTOFFS[1] - CUTOFFS[0] - 1)
    rel2 = jnp.clip(tsv - CUTOFFS[1], 0, CUTOFFS[2] - CUTOFFS[1] - 1)
    rel3 = jnp.clip(tsv - CUTOFFS[2], 0, 100000 - CUTOFFS[2] - 1)
    rel1_ref[...] = rel1
    rel2_ref[...] = rel2 // 2
    rel3_ref[...] = rel3 // 8
    q2_ref[...] = rel2 % 2
    q3_ref[...] = rel3 % 8


def _sc_gather(w21, w22, w23, r1, r2, r3):
    T = r1.shape[0]
    info = plsc.get_sparse_core_info()
    nc = info.num_cores
    nw = nc * info.num_subcores
    bpw = T // nw
    d1, d2, d3 = w21.shape[1], w22.shape[1], w23.shape[1]
    mesh = plsc.VectorSubcoreMesh(core_axis_name="c", subcore_axis_name="s")

    @functools.partial(
        pl.kernel, mesh=mesh,
        out_type=[
            jax.ShapeDtypeStruct((T, d1), jnp.float32),
            jax.ShapeDtypeStruct((T, d2), jnp.float32),
            jax.ShapeDtypeStruct((T, d3), jnp.float32),
        ],
        scratch_types=[
            pltpu.VMEM((bpw,), jnp.int32),
            pltpu.VMEM((bpw,), jnp.int32),
            pltpu.VMEM((bpw,), jnp.int32),
            pltpu.VMEM((bpw, d1), jnp.float32),
            pltpu.VMEM((bpw, d2), jnp.float32),
            pltpu.VMEM((bpw, d3), jnp.float32),
            pltpu.SemaphoreType.DMA,
        ],
    )
    def gather_k(t1, t2, t3, i1, i2, i3, g1, g2, g3,
                 iv1, iv2, iv3, rv1, rv2, rv3, sem):
        wid = jax.lax.axis_index("s") * nc + jax.lax.axis_index("c")
        base = wid * bpw
        for i_hbm, iv, t_hbm, rv, g_hbm in (
                (i1, iv1, t1, rv1, g1), (i2, iv2, t2, rv2, g2),
                (i3, iv3, t3, rv3, g3)):
            pltpu.sync_copy(i_hbm.at[pl.ds(base, bpw)], iv)
            pltpu.async_copy(t_hbm.at[iv], rv, sem).wait()
            pltpu.sync_copy(rv, g_hbm.at[pl.ds(base, bpw)])

    return gather_k(w21, w22, w23, r1, r2, r3)


def _tail_body(oc_ref, h_ref, w2_ref, out_ref, m_ref, s_ref,
               *, n_cols, blk, n_blk, n_tb):
    cb = pl.program_id(0)
    off = oc_ref[0]
    cnt = oc_ref[1]

    @pl.when(cb == 0)
    def _init():
        m_ref[...] = jnp.full(m_ref.shape, NEG_INF, jnp.float32)
        s_ref[...] = jnp.zeros(s_ref.shape, jnp.float32)

    def _update(masked):
        w2b = w2_ref[...].astype(jnp.bfloat16)
        for j in range(n_tb):
            row0 = j * TB
            active = (row0 < off + cnt) & (row0 + TB > off)

            @pl.when(active)
            def _step(row0=row0):
                hid = h_ref[row0:row0 + TB, :]
                logits = jax.lax.dot_general(
                    hid, w2b, (((1,), (1,)), ((), ())),
                    preferred_element_type=jnp.float32)
                if masked:
                    cols = cb * blk + jax.lax.broadcasted_iota(
                        jnp.int32, logits.shape, 1)
                    lm = jnp.where(cols < n_cols, logits, NEG_INF)
                else:
                    lm = logits
                bm = jnp.max(lm, axis=1, keepdims=True)
                m_old = m_ref[row0:row0 + TB, :]
                m_new = jnp.maximum(m_old, bm)
                s_ref[row0:row0 + TB, :] = (
                    s_ref[row0:row0 + TB, :] * jnp.exp(m_old - m_new)
                    + jnp.sum(jnp.exp(lm - m_new), axis=1, keepdims=True))
                m_ref[row0:row0 + TB, :] = m_new

    @pl.when(cb < n_blk - 1)
    def _full_blocks():
        _update(masked=False)

    @pl.when(cb == n_blk - 1)
    def _last_block():
        _update(masked=True)
        rows = jax.lax.broadcasted_iota(jnp.int32, (m_ref.shape[0], 1), 0)
        valid = (rows >= off) & (rows < off + cnt)
        out_ref[...] = jnp.where(valid, m_ref[...] + jnp.log(s_ref[...]), 0.0)


def _tail_logsumexp(offcnt, h_sorted, w2, blk):
    T = h_sorted.shape[0]
    n_cols, h = w2.shape
    n_blk = pl.cdiv(n_cols, blk)
    n_tb = T // TB
    body = functools.partial(_tail_body, n_cols=n_cols, blk=blk,
                             n_blk=n_blk, n_tb=n_tb)
    grid_spec = pltpu.PrefetchScalarGridSpec(
        num_scalar_prefetch=1,
        grid=(n_blk,),
        in_specs=[
            pl.BlockSpec((T, h), lambda cb, oc: (0, 0)),
            pl.BlockSpec((blk, h), lambda cb, oc: (cb, 0)),
        ],
        out_specs=pl.BlockSpec((T, 1), lambda cb, oc: (0, 0)),
        scratch_shapes=[
            pltpu.VMEM((T, 1), jnp.float32),
            pltpu.VMEM((T, 1), jnp.float32),
        ],
    )
    return pl.pallas_call(
        body,
        grid_spec=grid_spec,
        out_shape=jax.ShapeDtypeStruct((T, 1), jnp.float32),
    )(offcnt, h_sorted, w2)


def _tgt_body(oc_ref, h1_ref, h2_ref, h3_ref, g1_ref, g2_ref, g3_ref,
              q2_ref, q3_ref, out_ref):
    rows = jax.lax.broadcasted_iota(jnp.int32, out_ref.shape, 0)

    def dot(h_ref, g_ref, q_ref, gf):
        hf = h_ref[...].astype(jnp.float32)
        if gf == 1:
            return jnp.sum(g_ref[...] * hf, axis=1, keepdims=True)
        w = hf.shape[1]
        ht = jnp.concatenate([hf] * gf, axis=1)
        seg = jax.lax.broadcasted_iota(jnp.int32, ht.shape, 1) // w
        prod = jnp.where(seg == q_ref[...], g_ref[...] * ht, 0.0)
        return jnp.sum(prod, axis=1, keepdims=True)

    out = jnp.zeros(out_ref.shape, jnp.float32)
    for i, tgt in enumerate((dot(h1_ref, g1_ref, None, 1),
                             dot(h2_ref, g2_ref, q2_ref, 2),
                             dot(h3_ref, g3_ref, q3_ref, 8))):
        off = oc_ref[2 * i]
        cnt = oc_ref[2 * i + 1]
        out += jnp.where((rows >= off) & (rows < off + cnt), tgt, 0.0)
    out_ref[...] = out


def _tgt_logits(oc6, hs_list, gs_list, q2, q3):
    T = hs_list[0].shape[0]
    full = lambda a: pl.BlockSpec(a.shape, lambda i, oc: (0,) * a.ndim)
    grid_spec = pltpu.PrefetchScalarGridSpec(
        num_scalar_prefetch=1,
        grid=(1,),
        in_specs=[full(a) for a in (*hs_list, *gs_list, q2, q3)],
        out_specs=pl.BlockSpec((T, 1), lambda i, oc: (0, 0)),
    )
    return pl.pallas_call(
        _tgt_body,
        grid_spec=grid_spec,
        out_shape=jax.ShapeDtypeStruct((T, 1), jnp.float32),
    )(oc6, *hs_list, *gs_list, q2, q3)


def _head_body(x_ref, hw_ref, t_ref, pos_ref, l1_ref, l2_ref, l3_ref, tg_ref,
               out_ref, loss_ref, acc_ref, *, n_tb, n_tokens):
    tb = pl.program_id(0)
    logits = jax.lax.dot_general(
        x_ref[...].astype(jnp.bfloat16), hw_ref[...].astype(jnp.bfloat16),
        (((1,), (1,)), ((), ())),
        preferred_element_type=jnp.float32)
    cols = jax.lax.broadcasted_iota(jnp.int32, logits.shape, 1)
    m = jnp.max(logits, axis=1, keepdims=True)
    s = jnp.sum(jnp.exp(logits - m), axis=1, keepdims=True)
    t = t_ref[...]
    cl = _cluster_of(t)
    gidx = jnp.where(cl == 0, t, SHORTLIST + cl - 1)
    ht = (jnp.sum(jnp.where(cols == gidx, logits, 0.0), axis=1, keepdims=True)
          - m - jnp.log(s))
    lsum = tg_ref[...] - (l1_ref[...] + l2_ref[...] + l3_ref[...])
    p_rows = jax.lax.broadcasted_iota(jnp.int32, (lsum.shape[0], TB), 0)
    s_blk = (jnp.transpose(pos_ref[...]) == p_rows).astype(jnp.float32)
    local = jax.lax.dot_general(
        s_blk, lsum, (((0,), (0,)), ((), ())),
        preferred_element_type=jnp.float32)
    out = local + ht
    out_ref[...] = out

    @pl.when(tb == 0)
    def _z():
        acc_ref[0] = 0.0

    acc_ref[0] += jnp.sum(-out) / n_tokens

    @pl.when(tb == n_tb - 1)
    def _w():
        loss_ref[...] = jnp.full((1, 1), acc_ref[0], jnp.float32)


def kernel(myinput, target, head_W, W1_1, W2_1, W1_2, W2_2, W1_3, W2_3):
    x = myinput
    T, F = x.shape
    t2 = target.astype(jnp.int32).reshape(T, 1)
    n_tb = T // TB

    pos, cnt8, aoff8 = pl.pallas_call(
        functools.partial(_prep1_body, n_tokens=T),
        grid=(n_tb,),
        in_specs=[pl.BlockSpec((T, 1), lambda tb: (0, 0))],
        out_specs=[
            pl.BlockSpec((TB, 1), lambda tb: (tb, 0)),
            pl.BlockSpec((8, 1), lambda tb: (0, 0)),
            pl.BlockSpec((8, 1), lambda tb: (0, 0)),
        ],
        out_shape=[
            jax.ShapeDtypeStruct((T, 1), jnp.int32),
            jax.ShapeDtypeStruct((8, 1), jnp.int32),
            jax.ShapeDtypeStruct((8, 1), jnp.int32),
        ],
    )(t2)

    hs = [W1_1.shape[0], W1_2.shape[0], W1_3.shape[0]]
    h1s, h2s, h3s, rel1, rel2g, rel3g, q2, q3 = pl.pallas_call(
        functools.partial(_prep2_body, n_tokens=T),
        grid=(n_tb,),
        in_specs=[
            pl.BlockSpec((T, F), lambda tb: (0, 0)),
            pl.BlockSpec(W1_1.shape, lambda tb: (0, 0)),
            pl.BlockSpec(W1_2.shape, lambda tb: (0, 0)),
            pl.BlockSpec(W1_3.shape, lambda tb: (0, 0)),
            pl.BlockSpec((T, 1), lambda tb: (0, 0)),
            pl.BlockSpec((T, 1), lambda tb: (0, 0)),
        ],
        out_specs=[
            pl.BlockSpec((TB, hs[0]), lambda tb: (tb, 0)),
            pl.BlockSpec((TB, hs[1]), lambda tb: (tb, 0)),
            pl.BlockSpec((TB, hs[2]), lambda tb: (tb, 0)),
            pl.BlockSpec((TB, 1), lambda tb: (tb, 0)),
            pl.BlockSpec((TB, 1), lambda tb: (tb, 0)),
            pl.BlockSpec((TB, 1), lambda tb: (tb, 0)),
            pl.BlockSpec((TB, 1), lambda tb: (tb, 0)),
            pl.BlockSpec((TB, 1), lambda tb: (tb, 0)),
        ],
        out_shape=[
            jax.ShapeDtypeStruct((T, hs[0]), jnp.bfloat16),
            jax.ShapeDtypeStruct((T, hs[1]), jnp.bfloat16),
            jax.ShapeDtypeStruct((T, hs[2]), jnp.bfloat16),
            jax.ShapeDtypeStruct((T, 1), jnp.int32),
            jax.ShapeDtypeStruct((T, 1), jnp.int32),
            jax.ShapeDtypeStruct((T, 1), jnp.int32),
            jax.ShapeDtypeStruct((T, 1), jnp.int32),
            jax.ShapeDtypeStruct((T, 1), jnp.int32),
        ],
        scratch_shapes=[
            pltpu.VMEM((T, hs[0]), jnp.bfloat16),
            pltpu.VMEM((T, hs[1]), jnp.bfloat16),
            pltpu.VMEM((T, hs[2]), jnp.bfloat16),
        ],
    )(x, W1_1, W1_2, W1_3, t2, pos)

    oc1 = jnp.stack([aoff8[1, 0], cnt8[1, 0]])
    oc2 = jnp.stack([aoff8[2, 0], cnt8[2, 0]])
    oc3 = jnp.stack([aoff8[3, 0], cnt8[3, 0]])
    oc6 = jnp.concatenate([oc1, oc2, oc3])

    w22g = W2_2.reshape(-1, 128)
    w23g = W2_3.reshape(-1, 128)
    g1, g2, g3 = _sc_gather(W2_1, w22g, w23g, rel1.reshape(T),
                            rel2g.reshape(T), rel3g.reshape(T))

    l1 = _tail_logsumexp(oc1, h1s, W2_1, 2048)
    l2 = _tail_logsumexp(oc2, h2s, W2_2, 2048)
    l3 = _tail_logsumexp(oc3, h3s, W2_3, 2048)
    tg = _tgt_logits(oc6, [h1s, h2s, h3s], [g1, g2, g3], q2, q3)

    body = functools.partial(_head_body, n_tb=n_tb, n_tokens=T)
    out, loss = pl.pallas_call(
        body,
        grid=(n_tb,),
        in_specs=[
            pl.BlockSpec((TB, F), lambda tb: (tb, 0)),
            pl.BlockSpec(head_W.shape, lambda tb: (0, 0)),
            pl.BlockSpec((TB, 1), lambda tb: (tb, 0)),
            pl.BlockSpec((TB, 1), lambda tb: (tb, 0)),
            pl.BlockSpec((T, 1), lambda tb: (0, 0)),
            pl.BlockSpec((T, 1), lambda tb: (0, 0)),
            pl.BlockSpec((T, 1), lambda tb: (0, 0)),
            pl.BlockSpec((T, 1), lambda tb: (0, 0)),
        ],
        out_specs=[
            pl.BlockSpec((TB, 1), lambda tb: (tb, 0)),
            pl.BlockSpec((1, 1), lambda tb: (0, 0)),
        ],
        out_shape=[
            jax.ShapeDtypeStruct((T, 1), jnp.float32),
            jax.ShapeDtypeStruct((1, 1), jnp.float32),
        ],
        scratch_shapes=[pltpu.SMEM((1,), jnp.float32)],
    )(x, head_W, t2, pos, l1, l2, l3, tg)
    return (out.reshape(T), loss[0, 0])


# speedup vs baseline: 1.2462x; 1.0069x over previous
import functools

import jax
import jax.numpy as jnp
from jax.experimental import pallas as pl
from jax.experimental.pallas import tpu as pltpu
from jax.experimental.pallas import tpu_sc as plsc

CUTOFFS = [2000, 10000, 50000]
SHORTLIST = 2000
NEG_INF = float("-inf")
TB = 256


def _cluster_of(t):
    return ((t >= CUTOFFS[0]).astype(jnp.int32)
            + (t >= CUTOFFS[1]).astype(jnp.int32)
            + (t >= CUTOFFS[2]).astype(jnp.int32))


def _prep1_body(t_ref, pos_ref, cnt_ref, aoff_ref, *, n_tokens):
    tb = pl.program_id(0)
    t_all = t_ref[...]
    cl_all = _cluster_of(t_all)
    cl_row = jnp.transpose(cl_all)
    row0 = tb * TB
    cl_blk = _cluster_of(t_ref[pl.ds(row0, TB), :])
    cols = jax.lax.broadcasted_iota(jnp.int32, (TB, n_tokens), 1)
    rows = row0 + jax.lax.broadcasted_iota(jnp.int32, (TB, n_tokens), 0)
    rank = jnp.sum(jnp.where((cl_row == cl_blk) & (cols < rows), 1, 0),
                   axis=1, keepdims=True)
    cidx = jax.lax.broadcasted_iota(jnp.int32, (8, n_tokens), 0)
    cnt = jnp.sum((cl_row == cidx).astype(jnp.int32), axis=1, keepdims=True)
    rup = cnt
    c_r = jax.lax.broadcasted_iota(jnp.int32, (8, 8), 0)
    c_c = jax.lax.broadcasted_iota(jnp.int32, (8, 8), 1)
    aoff = jnp.sum(jnp.where(c_c < c_r, jnp.transpose(rup), 0), axis=1,
                   keepdims=True)
    cl8 = jax.lax.broadcasted_iota(jnp.int32, (TB, 8), 1)
    my_off = jnp.sum(jnp.where(cl_blk == cl8, jnp.transpose(aoff), 0),
                     axis=1, keepdims=True)
    pos_ref[...] = my_off + rank

    @pl.when(tb == 0)
    def _counts():
        cnt_ref[...] = cnt
        aoff_ref[...] = aoff


def _prep2_body(x_ref, w11_ref, w12_ref, w13_ref, t_ref, pos_ref,
                h1_ref, h2_ref, h3_ref, rel1_ref, rel2_ref, rel3_ref,
                q2_ref, q3_ref, hid1_ref, hid2_ref, hid3_ref, *, n_tokens):
    tb = pl.program_id(0)

    @pl.when(tb == 0)
    def _hidden():
        xb = x_ref[...].astype(jnp.bfloat16)
        for w_ref, hid_ref in ((w11_ref, hid1_ref), (w12_ref, hid2_ref),
                               (w13_ref, hid3_ref)):
            hid_ref[...] = jax.lax.dot_general(
                xb, w_ref[...].astype(jnp.bfloat16), (((1,), (1,)), ((), ())),
                preferred_element_type=jnp.float32).astype(jnp.bfloat16)

    row0 = tb * TB
    p_rows = row0 + jax.lax.broadcasted_iota(jnp.int32, (TB, n_tokens), 0)
    pos_row = jnp.transpose(pos_ref[...])
    s_blk = (pos_row == p_rows)
    s_bf = s_blk.astype(jnp.bfloat16)
    h1_ref[...] = jax.lax.dot_general(
        s_bf, hid1_ref[...], (((1,), (0,)), ((), ())),
        preferred_element_type=jnp.float32).astype(jnp.bfloat16)
    h2_ref[...] = jax.lax.dot_general(
        s_bf, hid2_ref[...], (((1,), (0,)), ((), ())),
        preferred_element_type=jnp.float32).astype(jnp.bfloat16)
    h3_ref[...] = jax.lax.dot_general(
        s_bf, hid3_ref[...], (((1,), (0,)), ((), ())),
        preferred_element_type=jnp.float32).astype(jnp.bfloat16)
    t_row = jnp.transpose(t_ref[...])
    tsv = jnp.sum(jnp.where(s_blk, t_row, 0), axis=1, keepdims=True)
    rel1 = jnp.clip(tsv - CUTOFFS[0], 0, CUTOFFS[1] - CUTOFFS[0] - 1)
    rel2 = jnp.clip(tsv - CUTOFFS[1], 0, CUTOFFS[2] - CUTOFFS[1] - 1)
    rel3 = jnp.clip(tsv - CUTOFFS[2], 0, 100000 - CUTOFFS[2] - 1)
    rel1_ref[...] = rel1
    rel2_ref[...] = rel2 // 2
    rel3_ref[...] = rel3 // 8
    q2_ref[...] = rel2 % 2
    q3_ref[...] = rel3 % 8


def _sc_gather(w21, w22, w23, r1, r2, r3):
    T = r1.shape[0]
    info = plsc.get_sparse_core_info()
    nc = info.num_cores
    nw = nc * info.num_subcores
    bpw = T // nw
    d1, d2, d3 = w21.shape[1], w22.shape[1], w23.shape[1]
    mesh = plsc.VectorSubcoreMesh(core_axis_name="c", subcore_axis_name="s")

    @functools.partial(
        pl.kernel, mesh=mesh,
        out_type=[
            jax.ShapeDtypeStruct((T, d1), jnp.float32),
            jax.ShapeDtypeStruct((T, d2), jnp.float32),
            jax.ShapeDtypeStruct((T, d3), jnp.float32),
        ],
        scratch_types=[
            pltpu.VMEM((bpw,), jnp.int32),
            pltpu.VMEM((bpw,), jnp.int32),
            pltpu.VMEM((bpw,), jnp.int32),
            pltpu.VMEM((bpw, d1), jnp.float32),
            pltpu.VMEM((bpw, d2), jnp.float32),
            pltpu.VMEM((bpw, d3), jnp.float32),
            pltpu.SemaphoreType.DMA,
            pltpu.SemaphoreType.DMA,
            pltpu.SemaphoreType.DMA,
        ],
    )
    def gather_k(t1, t2, t3, i1, i2, i3, g1, g2, g3,
                 iv1, iv2, iv3, rv1, rv2, rv3, sem1, sem2, sem3):
        wid = jax.lax.axis_index("s") * nc + jax.lax.axis_index("c")
        base = wid * bpw
        pltpu.sync_copy(i1.at[pl.ds(base, bpw)], iv1)
        pltpu.sync_copy(i2.at[pl.ds(base, bpw)], iv2)
        pltpu.sync_copy(i3.at[pl.ds(base, bpw)], iv3)
        c1 = pltpu.async_copy(t1.at[iv1], rv1, sem1)
        c2 = pltpu.async_copy(t2.at[iv2], rv2, sem2)
        c3 = pltpu.async_copy(t3.at[iv3], rv3, sem3)
        c1.wait()
        pltpu.sync_copy(rv1, g1.at[pl.ds(base, bpw)])
        c2.wait()
        pltpu.sync_copy(rv2, g2.at[pl.ds(base, bpw)])
        c3.wait()
        pltpu.sync_copy(rv3, g3.at[pl.ds(base, bpw)])

    return gather_k(w21, w22, w23, r1, r2, r3)


def _tail_body(oc_ref, h_ref, w2_ref, out_ref, m_ref, s_ref,
               *, n_cols, blk, n_blk, n_tb):
    cb = pl.program_id(0)
    off = oc_ref[0]
    cnt = oc_ref[1]

    @pl.when(cb == 0)
    def _init():
        m_ref[...] = jnp.full(m_ref.shape, NEG_INF, jnp.float32)
        s_ref[...] = jnp.zeros(s_ref.shape, jnp.float32)

    def _update(masked):
        w2b = w2_ref[...].astype(jnp.bfloat16)
        for j in range(n_tb):
            row0 = j * TB
            active = (row0 < off + cnt) & (row0 + TB > off)

            @pl.when(active)
            def _step(row0=row0):
                hid = h_ref[row0:row0 + TB, :]
                logits = jax.lax.dot_general(
                    hid, w2b, (((1,), (1,)), ((), ())),
                    preferred_element_type=jnp.float32)
                if masked:
                    cols = cb * blk + jax.lax.broadcasted_iota(
                        jnp.int32, logits.shape, 1)
                    lm = jnp.where(cols < n_cols, logits, NEG_INF)
                else:
                    lm = logits
                bm = jnp.max(lm, axis=1, keepdims=True)
                m_old = m_ref[row0:row0 + TB, :]
                m_new = jnp.maximum(m_old, bm)
                s_ref[row0:row0 + TB, :] = (
                    s_ref[row0:row0 + TB, :] * jnp.exp(m_old - m_new)
                    + jnp.sum(jnp.exp(lm - m_new), axis=1, keepdims=True))
                m_ref[row0:row0 + TB, :] = m_new

    @pl.when(cb < n_blk - 1)
    def _full_blocks():
        _update(masked=False)

    @pl.when(cb == n_blk - 1)
    def _last_block():
        _update(masked=True)
        rows = jax.lax.broadcasted_iota(jnp.int32, (m_ref.shape[0], 1), 0)
        valid = (rows >= off) & (rows < off + cnt)
        out_ref[...] = jnp.where(valid, m_ref[...] + jnp.log(s_ref[...]), 0.0)


def _tail_logsumexp(offcnt, h_sorted, w2, blk):
    T = h_sorted.shape[0]
    n_cols, h = w2.shape
    n_blk = pl.cdiv(n_cols, blk)
    n_tb = T // TB
    body = functools.partial(_tail_body, n_cols=n_cols, blk=blk,
                             n_blk=n_blk, n_tb=n_tb)
    grid_spec = pltpu.PrefetchScalarGridSpec(
        num_scalar_prefetch=1,
        grid=(n_blk,),
        in_specs=[
            pl.BlockSpec((T, h), lambda cb, oc: (0, 0)),
            pl.BlockSpec((blk, h), lambda cb, oc: (cb, 0)),
        ],
        out_specs=pl.BlockSpec((T, 1), lambda cb, oc: (0, 0)),
        scratch_shapes=[
            pltpu.VMEM((T, 1), jnp.float32),
            pltpu.VMEM((T, 1), jnp.float32),
        ],
    )
    return pl.pallas_call(
        body,
        grid_spec=grid_spec,
        out_shape=jax.ShapeDtypeStruct((T, 1), jnp.float32),
    )(offcnt, h_sorted, w2)


def _tgt_body(oc_ref, h1_ref, h2_ref, h3_ref, g1_ref, g2_ref, g3_ref,
              q2_ref, q3_ref, out_ref):
    rows = jax.lax.broadcasted_iota(jnp.int32, out_ref.shape, 0)

    def dot(h_ref, g_ref, q_ref, gf):
        hf = h_ref[...].astype(jnp.float32)
        if gf == 1:
            return jnp.sum(g_ref[...] * hf, axis=1, keepdims=True)
        w = hf.shape[1]
        ht = jnp.concatenate([hf] * gf, axis=1)
        seg = jax.lax.broadcasted_iota(jnp.int32, ht.shape, 1) // w
        prod = jnp.where(seg == q_ref[...], g_ref[...] * ht, 0.0)
        return jnp.sum(prod, axis=1, keepdims=True)

    out = jnp.zeros(out_ref.shape, jnp.float32)
    for i, tgt in enumerate((dot(h1_ref, g1_ref, None, 1),
                             dot(h2_ref, g2_ref, q2_ref, 2),
                             dot(h3_ref, g3_ref, q3_ref, 8))):
        off = oc_ref[2 * i]
        cnt = oc_ref[2 * i + 1]
        out += jnp.where((rows >= off) & (rows < off + cnt), tgt, 0.0)
    out_ref[...] = out


def _tgt_logits(oc6, hs_list, gs_list, q2, q3):
    T = hs_list[0].shape[0]
    full = lambda a: pl.BlockSpec(a.shape, lambda i, oc: (0,) * a.ndim)
    grid_spec = pltpu.PrefetchScalarGridSpec(
        num_scalar_prefetch=1,
        grid=(1,),
        in_specs=[full(a) for a in (*hs_list, *gs_list, q2, q3)],
        out_specs=pl.BlockSpec((T, 1), lambda i, oc: (0, 0)),
    )
    return pl.pallas_call(
        _tgt_body,
        grid_spec=grid_spec,
        out_shape=jax.ShapeDtypeStruct((T, 1), jnp.float32),
    )(oc6, *hs_list, *gs_list, q2, q3)


def _head_body(x_ref, hw_ref, t_ref, pos_ref, l1_ref, l2_ref, l3_ref, tg_ref,
               out_ref, loss_ref, acc_ref, *, n_tb, n_tokens):
    tb = pl.program_id(0)
    logits = jax.lax.dot_general(
        x_ref[...].astype(jnp.bfloat16), hw_ref[...].astype(jnp.bfloat16),
        (((1,), (1,)), ((), ())),
        preferred_element_type=jnp.float32)
    cols = jax.lax.broadcasted_iota(jnp.int32, logits.shape, 1)
    m = jnp.max(logits, axis=1, keepdims=True)
    s = jnp.sum(jnp.exp(logits - m), axis=1, keepdims=True)
    t = t_ref[...]
    cl = _cluster_of(t)
    gidx = jnp.where(cl == 0, t, SHORTLIST + cl - 1)
    ht = (jnp.sum(jnp.where(cols == gidx, logits, 0.0), axis=1, keepdims=True)
          - m - jnp.log(s))
    lsum = tg_ref[...] - (l1_ref[...] + l2_ref[...] + l3_ref[...])
    p_rows = jax.lax.broadcasted_iota(jnp.int32, (lsum.shape[0], TB), 0)
    s_blk = (jnp.transpose(pos_ref[...]) == p_rows).astype(jnp.float32)
    local = jax.lax.dot_general(
        s_blk, lsum, (((0,), (0,)), ((), ())),
        preferred_element_type=jnp.float32)
    out = local + ht
    out_ref[...] = out

    @pl.when(tb == 0)
    def _z():
        acc_ref[0] = 0.0

    acc_ref[0] += jnp.sum(-out) / n_tokens

    @pl.when(tb == n_tb - 1)
    def _w():
        loss_ref[...] = jnp.full((1, 1), acc_ref[0], jnp.float32)


def kernel(myinput, target, head_W, W1_1, W2_1, W1_2, W2_2, W1_3, W2_3):
    x = myinput
    T, F = x.shape
    t2 = target.astype(jnp.int32).reshape(T, 1)
    n_tb = T // TB

    pos, cnt8, aoff8 = pl.pallas_call(
        functools.partial(_prep1_body, n_tokens=T),
        grid=(n_tb,),
        in_specs=[pl.BlockSpec((T, 1), lambda tb: (0, 0))],
        out_specs=[
            pl.BlockSpec((TB, 1), lambda tb: (tb, 0)),
            pl.BlockSpec((8, 1), lambda tb: (0, 0)),
            pl.BlockSpec((8, 1), lambda tb: (0, 0)),
        ],
        out_shape=[
            jax.ShapeDtypeStruct((T, 1), jnp.int32),
            jax.ShapeDtypeStruct((8, 1), jnp.int32),
            jax.ShapeDtypeStruct((8, 1), jnp.int32),
        ],
    )(t2)

    hs = [W1_1.shape[0], W1_2.shape[0], W1_3.shape[0]]
    h1s, h2s, h3s, rel1, rel2g, rel3g, q2, q3 = pl.pallas_call(
        functools.partial(_prep2_body, n_tokens=T),
        grid=(n_tb,),
        in_specs=[
            pl.BlockSpec((T, F), lambda tb: (0, 0)),
            pl.BlockSpec(W1_1.shape, lambda tb: (0, 0)),
            pl.BlockSpec(W1_2.shape, lambda tb: (0, 0)),
            pl.BlockSpec(W1_3.shape, lambda tb: (0, 0)),
            pl.BlockSpec((T, 1), lambda tb: (0, 0)),
            pl.BlockSpec((T, 1), lambda tb: (0, 0)),
        ],
        out_specs=[
            pl.BlockSpec((TB, hs[0]), lambda tb: (tb, 0)),
            pl.BlockSpec((TB, hs[1]), lambda tb: (tb, 0)),
            pl.BlockSpec((TB, hs[2]), lambda tb: (tb, 0)),
            pl.BlockSpec((TB, 1), lambda tb: (tb, 0)),
            pl.BlockSpec((TB, 1), lambda tb: (tb, 0)),
            pl.BlockSpec((TB, 1), lambda tb: (tb, 0)),
            pl.BlockSpec((TB, 1), lambda tb: (tb, 0)),
            pl.BlockSpec((TB, 1), lambda tb: (tb, 0)),
        ],
        out_shape=[
            jax.ShapeDtypeStruct((T, hs[0]), jnp.bfloat16),
            jax.ShapeDtypeStruct((T, hs[1]), jnp.bfloat16),
            jax.ShapeDtypeStruct((T, hs[2]), jnp.bfloat16),
            jax.ShapeDtypeStruct((T, 1), jnp.int32),
            jax.ShapeDtypeStruct((T, 1), jnp.int32),
            jax.ShapeDtypeStruct((T, 1), jnp.int32),
            jax.ShapeDtypeStruct((T, 1), jnp.int32),
            jax.ShapeDtypeStruct((T, 1), jnp.int32),
        ],
        scratch_shapes=[
            pltpu.VMEM((T, hs[0]), jnp.bfloat16),
            pltpu.VMEM((T, hs[1]), jnp.bfloat16),
            pltpu.VMEM((T, hs[2]), jnp.bfloat16),
        ],
    )(x, W1_1, W1_2, W1_3, t2, pos)

    oc1 = jnp.stack([aoff8[1, 0], cnt8[1, 0]])
    oc2 = jnp.stack([aoff8[2, 0], cnt8[2, 0]])
    oc3 = jnp.stack([aoff8[3, 0], cnt8[3, 0]])
    oc6 = jnp.concatenate([oc1, oc2, oc3])

    w22g = W2_2.reshape(-1, 128)
    w23g = W2_3.reshape(-1, 128)
    g1, g2, g3 = _sc_gather(W2_1, w22g, w23g, rel1.reshape(T),
                            rel2g.reshape(T), rel3g.reshape(T))

    l1 = _tail_logsumexp(oc1, h1s, W2_1, 2048)
    l2 = _tail_logsumexp(oc2, h2s, W2_2, 2048)
    l3 = _tail_logsumexp(oc3, h3s, W2_3, 2048)
    tg = _tgt_logits(oc6, [h1s, h2s, h3s], [g1, g2, g3], q2, q3)

    body = functools.partial(_head_body, n_tb=n_tb, n_tokens=T)
    out, loss = pl.pallas_call(
        body,
        grid=(n_tb,),
        in_specs=[
            pl.BlockSpec((TB, F), lambda tb: (tb, 0)),
            pl.BlockSpec(head_W.shape, lambda tb: (0, 0)),
            pl.BlockSpec((TB, 1), lambda tb: (tb, 0)),
            pl.BlockSpec((TB, 1), lambda tb: (tb, 0)),
            pl.BlockSpec((T, 1), lambda tb: (0, 0)),
            pl.BlockSpec((T, 1), lambda tb: (0, 0)),
            pl.BlockSpec((T, 1), lambda tb: (0, 0)),
            pl.BlockSpec((T, 1), lambda tb: (0, 0)),
        ],
        out_specs=[
            pl.BlockSpec((TB, 1), lambda tb: (tb, 0)),
            pl.BlockSpec((1, 1), lambda tb: (0, 0)),
        ],
        out_shape=[
            jax.ShapeDtypeStruct((T, 1), jnp.float32),
            jax.ShapeDtypeStruct((1, 1), jnp.float32),
        ],
        scratch_shapes=[pltpu.SMEM((1,), jnp.float32)],
    )(x, head_W, t2, pos, l1, l2, l3, tg)
    return (out.reshape(T), loss[0, 0])
